# baseline (speedup 1.0000x reference)
import jax
import jax.numpy as jnp
from jax.experimental import pallas as pl

SEQ_LEN = 4096
FREQ_TOPK = 20


def _sub_kernel(x_ref, y_ref, o_ref):
    o_ref[...] = x_ref[...] - y_ref[...]


def kernel(batch_x, W1, b1, W2, b2, W3, b3):
    x = batch_x
    xf = jnp.fft.rfft(x, axis=1)
    mag = jnp.abs(xf)
    mag_t = jnp.moveaxis(mag, 1, -1)
    _, idx = jax.lax.top_k(mag_t, FREQ_TOPK)
    indices = jnp.moveaxis(idx, -1, 1)
    B, F, N = xf.shape
    b_idx = jnp.arange(B)[:, None, None]
    n_idx = jnp.arange(N)[None, None, :]
    mask = jnp.zeros(xf.shape, dtype=xf.dtype).at[b_idx, indices, n_idx].set(1)
    xf_filtered = xf * mask
    x_filtered = jnp.fft.irfft(xf_filtered, n=x.shape[1], axis=1).real.astype(jnp.float32)

    out = pl.pallas_call(
        _sub_kernel,
        out_shape=jax.ShapeDtypeStruct(x.shape, x.dtype),
        grid=(x.shape[0],),
        in_specs=[
            pl.BlockSpec((1, SEQ_LEN, x.shape[2]), lambda b: (b, 0, 0)),
            pl.BlockSpec((1, SEQ_LEN, x.shape[2]), lambda b: (b, 0, 0)),
        ],
        out_specs=pl.BlockSpec((1, SEQ_LEN, x.shape[2]), lambda b: (b, 0, 0)),
    )(x, x_filtered)
    return out


# speedup vs baseline: 12.4825x; 12.4825x over previous
import numpy as np
import jax
import jax.numpy as jnp
from jax.experimental import pallas as pl
from jax.experimental.pallas import tpu as pltpu

T = 4096
NCH = 256
FREQ = T // 2 + 1
FP = 2304
TOPK = 20
CHUNK = 128
NCHUNK = FP // CHUNK


def _basis_np():
    f = np.arange(FP, dtype=np.float64)
    t = np.arange(T, dtype=np.float64)
    ft = np.outer(f, t)
    ang = np.mod(ft, T) * (2.0 * np.pi / T)
    c = np.cos(ang)
    s = np.sin(ang)
    c[FREQ:, :] = 0.0
    s[FREQ:, :] = 0.0
    w = np.full((FP, 1), 2.0 / T)
    w[0, 0] = 1.0 / T
    w[T // 2, 0] = 1.0 / T
    w[FREQ:, 0] = 0.0
    ct = c.astype(np.float32)
    st = s.astype(np.float32)
    ic = np.ascontiguousarray((c * w).T).astype(jnp.bfloat16)
    isn = np.ascontiguousarray((s * w).T).astype(jnp.bfloat16)
    return ct, st, ic, isn


_CT, _ST, _IC, _ISN = _basis_np()


def _fwd_kernel(ct_ref, st_ref, x_ref, xre_ref, xim_ref):
    xb = x_ref[0]
    xre_ref[0] = jnp.dot(ct_ref[...], xb, preferred_element_type=jnp.float32,
                         precision=jax.lax.Precision.HIGHEST)
    xim_ref[0] = jnp.dot(st_ref[...], xb, preferred_element_type=jnp.float32,
                         precision=jax.lax.Precision.HIGHEST)


def _topk_kernel(xre_ref, xim_ref, yre_ref, yim_ref, mag_scr):
    for c in range(NCHUNK):
        sl = pl.ds(c * CHUNK, CHUNK)
        xr = xre_ref[0, sl, :]
        xi = xim_ref[0, sl, :]
        mag_scr[sl, :] = xr * xr + xi * xi

    m = jnp.max(mag_scr[pl.ds(0, CHUNK), :], axis=0, keepdims=True)
    for c in range(1, NCHUNK):
        m = jnp.maximum(
            m, jnp.max(mag_scr[pl.ds(c * CHUNK, CHUNK), :], axis=0, keepdims=True))

    def body(_, m):
        nm = jnp.full((1, NCH), -2.0, dtype=jnp.float32)
        for c in range(NCHUNK):
            sl = pl.ds(c * CHUNK, CHUNK)
            mg = mag_scr[sl, :]
            pick = mg >= m
            mgu = jnp.where(pick, -1.0, mg)
            mag_scr[sl, :] = mgu
            nm = jnp.maximum(nm, jnp.max(mgu, axis=0, keepdims=True))
        return nm

    jax.lax.fori_loop(0, TOPK, body, m)

    for c in range(NCHUNK):
        sl = pl.ds(c * CHUNK, CHUNK)
        ws = jnp.where(mag_scr[sl, :] < 0.0, 1.0, 0.0)
        yre_ref[0, sl, :] = (xre_ref[0, sl, :] * ws).astype(jnp.bfloat16)
        yim_ref[0, sl, :] = (xim_ref[0, sl, :] * ws).astype(jnp.bfloat16)


def _inv_kernel(ic_ref, isn_ref, yre_ref, yim_ref, x_ref, o_ref):
    filt = jnp.dot(ic_ref[...], yre_ref[0], preferred_element_type=jnp.float32)
    filt = filt + jnp.dot(isn_ref[...], yim_ref[0], preferred_element_type=jnp.float32)
    o_ref[0] = x_ref[0] - filt


def kernel(batch_x, W1, b1, W2, b2, W3, b3):
    B = batch_x.shape[0]
    ct = jnp.asarray(_CT)
    st = jnp.asarray(_ST)
    ic = jnp.asarray(_IC)
    isn = jnp.asarray(_ISN)

    FH = FP // 4
    xre, xim = pl.pallas_call(
        _fwd_kernel,
        grid=(4, B),
        in_specs=[
            pl.BlockSpec((FH, T), lambda h, b: (h, 0)),
            pl.BlockSpec((FH, T), lambda h, b: (h, 0)),
            pl.BlockSpec((1, T, NCH), lambda h, b: (b, 0, 0)),
        ],
        out_specs=[
            pl.BlockSpec((1, FH, NCH), lambda h, b: (b, h, 0)),
            pl.BlockSpec((1, FH, NCH), lambda h, b: (b, h, 0)),
        ],
        out_shape=[jax.ShapeDtypeStruct((B, FP, NCH), jnp.float32)] * 2,
    )(ct, st, batch_x)

    yre, yim = pl.pallas_call(
        _topk_kernel,
        grid=(B,),
        in_specs=[
            pl.BlockSpec((1, FP, NCH), lambda b: (b, 0, 0)),
            pl.BlockSpec((1, FP, NCH), lambda b: (b, 0, 0)),
        ],
        out_specs=[
            pl.BlockSpec((1, FP, NCH), lambda b: (b, 0, 0)),
            pl.BlockSpec((1, FP, NCH), lambda b: (b, 0, 0)),
        ],
        out_shape=[jax.ShapeDtypeStruct((B, FP, NCH), jnp.bfloat16)] * 2,
        scratch_shapes=[pltpu.VMEM((FP, NCH), jnp.float32)],
    )(xre, xim)

    TH = T // 4
    out = pl.pallas_call(
        _inv_kernel,
        grid=(4, B),
        in_specs=[
            pl.BlockSpec((TH, FP), lambda h, b: (h, 0)),
            pl.BlockSpec((TH, FP), lambda h, b: (h, 0)),
            pl.BlockSpec((1, FP, NCH), lambda h, b: (b, 0, 0)),
            pl.BlockSpec((1, FP, NCH), lambda h, b: (b, 0, 0)),
            pl.BlockSpec((1, TH, NCH), lambda h, b: (b, h, 0)),
        ],
        out_specs=pl.BlockSpec((1, TH, NCH), lambda h, b: (b, h, 0)),
        out_shape=jax.ShapeDtypeStruct((B, T, NCH), jnp.float32),
    )(ic, isn, yre, yim, batch_x)
    return out


# speedup vs baseline: 20.8708x; 1.6720x over previous
import numpy as np
import jax
import jax.numpy as jnp
from jax.experimental import pallas as pl
from jax.experimental.pallas import tpu as pltpu

T = 4096
NCH = 256
FREQ = T // 2 + 1
FP = 2304
TOPK = 20
CHUNK = 128
NCHUNK = FP // CHUNK


def _basis_np():
    f = np.arange(FP, dtype=np.float64)
    t = np.arange(T, dtype=np.float64)
    ft = np.outer(f, t)
    ang = np.mod(ft, T) * (2.0 * np.pi / T)
    c = np.cos(ang)
    s = np.sin(ang)
    c[FREQ:, :] = 0.0
    s[FREQ:, :] = 0.0
    w = np.full((FP, 1), 2.0 / T)
    w[0, 0] = 1.0 / T
    w[T // 2, 0] = 1.0 / T
    w[FREQ:, 0] = 0.0
    ct = c.astype(np.float32)
    st = s.astype(np.float32)
    cth = ct.astype(jnp.bfloat16)
    ctl = (ct - np.asarray(cth, np.float32)).astype(jnp.bfloat16)
    sth = st.astype(jnp.bfloat16)
    stl = (st - np.asarray(sth, np.float32)).astype(jnp.bfloat16)
    ic = np.ascontiguousarray((c * w).T).astype(jnp.bfloat16)
    isn = np.ascontiguousarray((s * w).T).astype(jnp.bfloat16)
    return cth, ctl, sth, stl, ic, isn


_CTH, _CTL, _STH, _STL, _IC, _ISN = _basis_np()


def _fwd_kernel(cth_ref, ctl_ref, sth_ref, stl_ref, x_ref, xre_ref, xim_ref):
    xb = x_ref[0]
    xh = xb.astype(jnp.bfloat16)
    xl = (xb - xh.astype(jnp.float32)).astype(jnp.bfloat16)

    def mm3(h_ref, l_ref):
        acc = jnp.dot(h_ref[...], xh, preferred_element_type=jnp.float32)
        acc += jnp.dot(h_ref[...], xl, preferred_element_type=jnp.float32)
        acc += jnp.dot(l_ref[...], xh, preferred_element_type=jnp.float32)
        return acc

    xre_ref[0] = mm3(cth_ref, ctl_ref)
    xim_ref[0] = mm3(sth_ref, stl_ref)


def _topk_kernel(xre_ref, xim_ref, yre_ref, yim_ref, mag_scr):
    for c in range(NCHUNK):
        sl = pl.ds(c * CHUNK, CHUNK)
        xr = xre_ref[0, sl, :]
        xi = xim_ref[0, sl, :]
        mag_scr[sl, :] = xr * xr + xi * xi

    m = jnp.max(mag_scr[pl.ds(0, CHUNK), :], axis=0, keepdims=True)
    for c in range(1, NCHUNK):
        m = jnp.maximum(
            m, jnp.max(mag_scr[pl.ds(c * CHUNK, CHUNK), :], axis=0, keepdims=True))

    def body(_, m):
        nm = jnp.full((1, NCH), -2.0, dtype=jnp.float32)
        for c in range(NCHUNK):
            sl = pl.ds(c * CHUNK, CHUNK)
            mg = mag_scr[sl, :]
            pick = mg >= m
            mgu = jnp.where(pick, -1.0, mg)
            mag_scr[sl, :] = mgu
            nm = jnp.maximum(nm, jnp.max(mgu, axis=0, keepdims=True))
        return nm

    jax.lax.fori_loop(0, TOPK, body, m)

    for c in range(NCHUNK):
        sl = pl.ds(c * CHUNK, CHUNK)
        ws = jnp.where(mag_scr[sl, :] < 0.0, 1.0, 0.0)
        yre_ref[0, sl, :] = (xre_ref[0, sl, :] * ws).astype(jnp.bfloat16)
        yim_ref[0, sl, :] = (xim_ref[0, sl, :] * ws).astype(jnp.bfloat16)


def _inv_kernel(ic_ref, isn_ref, yre_ref, yim_ref, x_ref, o_ref):
    filt = jnp.dot(ic_ref[...], yre_ref[0], preferred_element_type=jnp.float32)
    filt = filt + jnp.dot(isn_ref[...], yim_ref[0], preferred_element_type=jnp.float32)
    o_ref[0] = x_ref[0] - filt


def kernel(batch_x, W1, b1, W2, b2, W3, b3):
    B = batch_x.shape[0]
    cth = jnp.asarray(_CTH)
    ctl = jnp.asarray(_CTL)
    sth = jnp.asarray(_STH)
    stl = jnp.asarray(_STL)
    ic = jnp.asarray(_IC)
    isn = jnp.asarray(_ISN)

    FH = FP // 4
    xre, xim = pl.pallas_call(
        _fwd_kernel,
        grid=(4, B),
        in_specs=[
            pl.BlockSpec((FH, T), lambda h, b: (h, 0)),
            pl.BlockSpec((FH, T), lambda h, b: (h, 0)),
            pl.BlockSpec((FH, T), lambda h, b: (h, 0)),
            pl.BlockSpec((FH, T), lambda h, b: (h, 0)),
            pl.BlockSpec((1, T, NCH), lambda h, b: (b, 0, 0)),
        ],
        out_specs=[
            pl.BlockSpec((1, FH, NCH), lambda h, b: (b, h, 0)),
            pl.BlockSpec((1, FH, NCH), lambda h, b: (b, h, 0)),
        ],
        out_shape=[jax.ShapeDtypeStruct((B, FP, NCH), jnp.float32)] * 2,
    )(cth, ctl, sth, stl, batch_x)

    yre, yim = pl.pallas_call(
        _topk_kernel,
        grid=(B,),
        in_specs=[
            pl.BlockSpec((1, FP, NCH), lambda b: (b, 0, 0)),
            pl.BlockSpec((1, FP, NCH), lambda b: (b, 0, 0)),
        ],
        out_specs=[
            pl.BlockSpec((1, FP, NCH), lambda b: (b, 0, 0)),
            pl.BlockSpec((1, FP, NCH), lambda b: (b, 0, 0)),
        ],
        out_shape=[jax.ShapeDtypeStruct((B, FP, NCH), jnp.bfloat16)] * 2,
        scratch_shapes=[pltpu.VMEM((FP, NCH), jnp.float32)],
    )(xre, xim)

    TH = T // 4
    out = pl.pallas_call(
        _inv_kernel,
        grid=(4, B),
        in_specs=[
            pl.BlockSpec((TH, FP), lambda h, b: (h, 0)),
            pl.BlockSpec((TH, FP), lambda h, b: (h, 0)),
            pl.BlockSpec((1, FP, NCH), lambda h, b: (b, 0, 0)),
            pl.BlockSpec((1, FP, NCH), lambda h, b: (b, 0, 0)),
            pl.BlockSpec((1, TH, NCH), lambda h, b: (b, h, 0)),
        ],
        out_specs=pl.BlockSpec((1, TH, NCH), lambda h, b: (b, h, 0)),
        out_shape=jax.ShapeDtypeStruct((B, T, NCH), jnp.float32),
    )(ic, isn, yre, yim, batch_x)
    return out
